# baseline (speedup 1.0000x reference)
import functools
import jax
import jax.numpy as jnp
from jax import lax
from jax.experimental import pallas as pl
from jax.experimental.pallas import tpu as pltpu, tpu_sc as plsc

CH = 32


def kernel(inputs, table):
    b, l = inputs.shape
    d = table.shape[1]
    nw = 32
    rpw = l // nw
    nch = rpw // CH
    mesh = plsc.VectorSubcoreMesh(core_axis_name="c", subcore_axis_name="s")

    @functools.partial(
        pl.kernel,
        mesh=mesh,
        out_type=jax.ShapeDtypeStruct((b, l, d), table.dtype),
        scratch_types=[
            pltpu.VMEM((CH, d), table.dtype),
            pltpu.VMEM((CH, d), table.dtype),
            pltpu.SemaphoreType.DMA((2,)),
            pltpu.SemaphoreType.DMA((2,)),
        ],
    )
    def k(table_hbm, out_hbm, buf0, buf1, sin, sout):
        bufs = (buf0, buf1)
        wid = lax.axis_index("s") * 2 + lax.axis_index("c")
        base = wid * rpw

        def start_in(c):
            return pltpu.async_copy(
                table_hbm.at[pl.ds(base + c * CH, CH)], bufs[c % 2], sin.at[c % 2]
            )

        in_cp = [None] * nch
        out_cps = [None] * nch
        in_cp[0] = start_in(0)
        for c in range(nch):
            p = c % 2
            in_cp[c].wait()
            if c + 1 < nch:
                if c - 1 >= 0:
                    for cp in out_cps[c - 1]:
                        cp.wait()
                in_cp[c + 1] = start_in(c + 1)
            out_cps[c] = [
                pltpu.async_copy(
                    bufs[p], out_hbm.at[bi, pl.ds(base + c * CH, CH)], sout.at[p]
                )
                for bi in range(b)
            ]
        for cc in range(max(nch - 2, 0), nch):
            for cp in out_cps[cc]:
                cp.wait()

    return k(table)
